# Initial kernel scaffold; baseline (speedup 1.0000x reference)
import jax
import jax.numpy as jnp
from jax.experimental import pallas as pl


def kernel(x, edge_index, edge_weight, bn_gamma, bn_beta, bn_mean, bn_var, W0, W1, fc1_w, fc1_b, fc2_w, fc2_b):
    raise NotImplementedError("write your pallas kernel here")


# speedup vs baseline: 1.1277x; 1.1277x over previous
import functools

import jax
import jax.numpy as jnp
from jax import lax
from jax.experimental import pallas as pl
from jax.experimental.pallas import tpu as pltpu
from jax.experimental.pallas import tpu_sc as plsc

_NC = 2
_NS = 16
_CHUNK = 128


def _prep_body(x_ref, scale_ref, shift_ref, wlo_ref, whi_ref, t_ref):
    n = x_ref.shape[0]
    xb = x_ref[...] * scale_ref[...] + shift_ref[...]
    t_ref[:n, :] = jnp.dot(xb, wlo_ref[...], preferred_element_type=jnp.float32)
    t_ref[n:, :] = jnp.dot(xb, whi_ref[...], preferred_element_type=jnp.float32)


def _prep(x, scale, shift, wlo, whi):
    n = x.shape[0]
    hid = wlo.shape[1]
    return pl.pallas_call(
        _prep_body,
        out_shape=jax.ShapeDtypeStruct((2 * n, hid), jnp.float32),
    )(x, scale, shift, wlo, whi)


def _sc_agg_call(n_nodes, hid, e_pad):
    ep = e_pad // _NS
    k_chunks = ep // _CHUNK
    half = hid // 2
    n_acc = -(-n_nodes // (_NS * 8)) * (_NS * 8)
    rows_per_tile = n_acc // _NS
    mesh = plsc.VectorSubcoreMesh(
        core_axis_name="c", subcore_axis_name="s",
        num_cores=_NC, num_subcores=_NS)

    @functools.partial(
        pl.kernel,
        out_type=jax.ShapeDtypeStruct((_NC, n_acc, half), jnp.float32),
        mesh=mesh,
        compiler_params=pltpu.CompilerParams(use_tc_tiling_on_sc=False),
        scratch_types=[
            pltpu.VMEM_SHARED((n_acc, half), jnp.float32),
            pltpu.VMEM((ep,), jnp.int32),
            pltpu.VMEM((ep,), jnp.int32),
            pltpu.VMEM((_CHUNK,), jnp.int32),
            pltpu.VMEM((_CHUNK,), jnp.int32),
            pltpu.VMEM((32 * _CHUNK,), jnp.float32),
            pltpu.VMEM((_CHUNK, hid), jnp.float32),
            pltpu.VMEM((_CHUNK, half), jnp.float32),
            pltpu.VMEM((256, half), jnp.float32),
            pltpu.SemaphoreType.DMA,
        ],
    )
    def sc_agg(src_hbm, dst_hbm, wr_hbm, t_hbm, out_hbm,
               acc_sh, src_v, dst_v, srcc_v, dstc_v, wr_v, rows_v, msg_v,
               zbuf_v, sem):
        cid = lax.axis_index("c")
        sid = lax.axis_index("s")
        base0 = sid * ep
        row_off = cid * n_nodes
        zeros16 = jnp.zeros((16,), jnp.float32)

        @pl.loop(0, 256)
        def _zrow(r):
            for q in range(half // 16):
                zbuf_v[r, pl.ds(q * 16, 16)] = zeros16

        zoff = 0
        for zp in (256, 256, 128):
            pltpu.sync_copy(
                zbuf_v.at[pl.ds(0, zp)],
                acc_sh.at[pl.ds(sid * rows_per_tile + zoff, zp)])
            zoff += zp
        plsc.subcore_barrier()

        pltpu.sync_copy(src_hbm.at[pl.ds(base0, ep)], src_v)
        pltpu.sync_copy(dst_hbm.at[pl.ds(base0, ep)], dst_v)

        @pl.loop(0, k_chunks)
        def _chunk(k):
            lo = k * _CHUNK
            pltpu.sync_copy(wr_hbm.at[pl.ds((base0 + lo) * 32, 32 * _CHUNK)],
                            wr_v)
            for g in range(_CHUNK // 16):
                srcc_v[pl.ds(g * 16, 16)] = (
                    src_v[pl.ds(lo + g * 16, 16)] + row_off)
                dstc_v[pl.ds(g * 16, 16)] = dst_v[pl.ds(lo + g * 16, 16)]

            pltpu.async_copy(t_hbm.at[srcc_v], rows_v, sem).wait()

            @pl.loop(0, _CHUNK)
            def _edge(e):
                mv = wr_v[pl.ds(e * 32, 16)]
                wv = wr_v[pl.ds(e * 32 + 16, 16)]
                for q in range(half // 16):
                    msg_v[e, pl.ds(q * 16, 16)] = (
                        mv * rows_v[e, pl.ds(q * 16, 16)]
                        + wv * rows_v[e, pl.ds(half + q * 16, 16)])

            pltpu.sync_copy(msg_v, acc_sh.at[dstc_v], add=True)

        plsc.subcore_barrier()
        pltpu.sync_copy(acc_sh.at[pl.ds(sid * rows_per_tile, rows_per_tile)],
                        out_hbm.at[cid, pl.ds(sid * rows_per_tile,
                                              rows_per_tile)])

    return sc_agg


def _relu_body(acc_ref, h_ref):
    h_ref[...] = jnp.maximum(acc_ref[...], 0.0)


def _relu(acc):
    return pl.pallas_call(
        _relu_body,
        out_shape=jax.ShapeDtypeStruct(acc.shape, jnp.float32),
    )(acc)


def _fc_body(nblk, hf_ref, w1_ref, b1_ref, w2_ref, b2_ref, out_ref, acc_ref):
    i = pl.program_id(0)

    @pl.when(i == 0)
    def _():
        acc_ref[...] = jnp.zeros_like(acc_ref)

    acc_ref[...] += jnp.dot(w1_ref[...], hf_ref[...],
                            preferred_element_type=jnp.float32)

    @pl.when(i == nblk - 1)
    def _():
        h2 = jnp.maximum(acc_ref[...] + b1_ref[...], 0.0)
        out_ref[...] = lax.dot_general(
            h2, w2_ref[...], (((0,), (1,)), ((), ())),
            preferred_element_type=jnp.float32) + b2_ref[...]


def _fc(hf, fc1_w, fc1_b, fc2_w, fc2_b, nblk=20):
    f1, kdim = fc1_w.shape
    ncls = fc2_w.shape[0]
    bk = kdim // nblk
    return pl.pallas_call(
        functools.partial(_fc_body, nblk),
        grid=(nblk,),
        in_specs=[
            pl.BlockSpec((bk, 1), lambda i: (i, 0)),
            pl.BlockSpec((f1, bk), lambda i: (0, i)),
            pl.BlockSpec((f1, 1), lambda i: (0, 0)),
            pl.BlockSpec((ncls, f1), lambda i: (0, 0)),
            pl.BlockSpec((1, ncls), lambda i: (0, 0)),
        ],
        out_specs=pl.BlockSpec((1, ncls), lambda i: (0, 0)),
        out_shape=jax.ShapeDtypeStruct((1, ncls), jnp.float32),
        scratch_shapes=[pltpu.VMEM((f1, 1), jnp.float32)],
    )(hf, fc1_w, fc1_b, fc2_w, fc2_b)


def kernel(x, edge_index, edge_weight, bn_gamma, bn_beta, bn_mean, bn_var,
           W0, W1, fc1_w, fc1_b, fc2_w, fc2_b):
    n, _ = x.shape
    hid = W0.shape[1]
    half = hid // 2
    scale = bn_gamma / jnp.sqrt(bn_var + 1e-5)
    shift = bn_beta - bn_mean * scale
    wlo = jnp.concatenate([W0[:, :half], W1[:, :half]], axis=1)
    whi = jnp.concatenate([W0[:, half:], W1[:, half:]], axis=1)
    t = _prep(x, scale.reshape(1, -1), shift.reshape(1, -1), wlo, whi)

    src = edge_index[0]
    dst = edge_index[1]
    e = src.shape[0]
    e_pad = e + (-e) % (_NS * _CHUNK)
    pad = e_pad - e
    src_p = jnp.concatenate([src, jnp.zeros((pad,), jnp.int32)])
    dst_p = jnp.concatenate([dst, jnp.ones((pad,), jnp.int32)])
    ew_p = jnp.concatenate([edge_weight, jnp.zeros((pad,), jnp.float32)])
    wm = (src_p == dst_p).astype(jnp.float32)
    wr = jnp.broadcast_to(
        jnp.stack([wm, ew_p], axis=1)[:, :, None],
        (e_pad, 2, 16)).reshape(e_pad * 32)

    acc = _sc_agg_call(n, hid, e_pad)(src_p, dst_p, wr, t)
    h = _relu(acc[:, :n, :])
    hf = jnp.concatenate([h[0], h[1]], axis=1).reshape(n * hid, 1)
    out = _fc(hf, fc1_w, fc1_b.reshape(-1, 1), fc2_w, fc2_b.reshape(1, -1))
    return out


# speedup vs baseline: 1.1551x; 1.0243x over previous
import functools

import jax
import jax.numpy as jnp
from jax import lax
from jax.experimental import pallas as pl
from jax.experimental.pallas import tpu as pltpu
from jax.experimental.pallas import tpu_sc as plsc

_NC = 2
_NS = 16
_CHUNK = 128


def _prep_body(x_ref, scale_ref, shift_ref, wlo_ref, whi_ref, t_ref):
    n = x_ref.shape[0]
    xb = x_ref[...] * scale_ref[...] + shift_ref[...]
    t_ref[:n, :] = jnp.dot(xb, wlo_ref[...], preferred_element_type=jnp.float32)
    t_ref[n:, :] = jnp.dot(xb, whi_ref[...], preferred_element_type=jnp.float32)


def _prep(x, scale, shift, wlo, whi):
    n = x.shape[0]
    hid = wlo.shape[1]
    return pl.pallas_call(
        _prep_body,
        out_shape=jax.ShapeDtypeStruct((2 * n, hid), jnp.float32),
    )(x, scale, shift, wlo, whi)


def _sc_agg_call(n_nodes, hid, e_pad):
    ep = e_pad // _NS
    k_chunks = ep // _CHUNK
    half = hid // 2
    n_acc = -(-n_nodes // (_NS * 8)) * (_NS * 8)
    rows_per_tile = n_acc // _NS
    mesh = plsc.VectorSubcoreMesh(
        core_axis_name="c", subcore_axis_name="s",
        num_cores=_NC, num_subcores=_NS)

    @functools.partial(
        pl.kernel,
        out_type=jax.ShapeDtypeStruct((_NC, n_acc, half), jnp.float32),
        mesh=mesh,
        compiler_params=pltpu.CompilerParams(use_tc_tiling_on_sc=False),
        scratch_types=[
            pltpu.VMEM_SHARED((n_acc, half), jnp.float32),
            pltpu.VMEM((ep,), jnp.int32),
            pltpu.VMEM((ep,), jnp.int32),
            pltpu.VMEM((_CHUNK,), jnp.int32),
            pltpu.VMEM((_CHUNK,), jnp.int32),
            pltpu.VMEM((8 * 32 * _CHUNK,), jnp.float32),
            pltpu.VMEM((_CHUNK, hid), jnp.float32),
            pltpu.VMEM((_CHUNK, half), jnp.float32),
            pltpu.VMEM((256, half), jnp.float32),
            pltpu.SemaphoreType.DMA,
        ],
    )
    def sc_agg(src_hbm, dst_hbm, wr_hbm, t_hbm, out_hbm,
               acc_sh, src_v, dst_v, srcc_v, dstc_v, wr_v, rows_v, msg_v,
               zbuf_v, sem):
        cid = lax.axis_index("c")
        sid = lax.axis_index("s")
        base0 = sid * ep
        row_off = cid * n_nodes
        zeros16 = jnp.zeros((16,), jnp.float32)

        @pl.loop(0, 256)
        def _zrow(r):
            for q in range(half // 16):
                zbuf_v[r, pl.ds(q * 16, 16)] = zeros16

        zoff = 0
        for zp in (256, 256, 128):
            pltpu.sync_copy(
                zbuf_v.at[pl.ds(0, zp)],
                acc_sh.at[pl.ds(sid * rows_per_tile + zoff, zp)])
            zoff += zp
        plsc.subcore_barrier()

        pltpu.sync_copy(src_hbm.at[pl.ds(base0, ep)], src_v)
        pltpu.sync_copy(dst_hbm.at[pl.ds(base0, ep)], dst_v)

        @pl.loop(0, k_chunks)
        def _chunk(k):
            lo = k * _CHUNK
            @pl.when(lax.rem(k, 8) == 0)
            def _():
                pltpu.sync_copy(
                    wr_hbm.at[pl.ds((base0 + lo) * 32, 8 * 32 * _CHUNK)],
                    wr_v)
            woff = lax.rem(k, 8) * (32 * _CHUNK)
            for g in range(_CHUNK // 16):
                srcc_v[pl.ds(g * 16, 16)] = (
                    src_v[pl.ds(lo + g * 16, 16)] + row_off)
                dstc_v[pl.ds(g * 16, 16)] = dst_v[pl.ds(lo + g * 16, 16)]

            pltpu.async_copy(t_hbm.at[srcc_v], rows_v, sem).wait()

            @pl.loop(0, _CHUNK, unroll=4)
            def _edge(e):
                mv = wr_v[pl.ds(woff + e * 32, 16)]
                wv = wr_v[pl.ds(woff + e * 32 + 16, 16)]
                for q in range(half // 16):
                    msg_v[e, pl.ds(q * 16, 16)] = (
                        mv * rows_v[e, pl.ds(q * 16, 16)]
                        + wv * rows_v[e, pl.ds(half + q * 16, 16)])

            pltpu.sync_copy(msg_v, acc_sh.at[dstc_v], add=True)

        plsc.subcore_barrier()
        pltpu.sync_copy(acc_sh.at[pl.ds(sid * rows_per_tile, rows_per_tile)],
                        out_hbm.at[cid, pl.ds(sid * rows_per_tile,
                                              rows_per_tile)])

    return sc_agg


def _relu_body(acc_ref, h_ref):
    h_ref[...] = jnp.maximum(acc_ref[...], 0.0)


def _relu(acc):
    return pl.pallas_call(
        _relu_body,
        out_shape=jax.ShapeDtypeStruct(acc.shape, jnp.float32),
    )(acc)


def _fc_body(nblk, hf_ref, w1_ref, b1_ref, w2_ref, b2_ref, out_ref, acc_ref):
    i = pl.program_id(0)

    @pl.when(i == 0)
    def _():
        acc_ref[...] = jnp.zeros_like(acc_ref)

    acc_ref[...] += jnp.dot(w1_ref[...], hf_ref[...],
                            preferred_element_type=jnp.float32)

    @pl.when(i == nblk - 1)
    def _():
        h2 = jnp.maximum(acc_ref[...] + b1_ref[...], 0.0)
        out_ref[...] = lax.dot_general(
            h2, w2_ref[...], (((0,), (1,)), ((), ())),
            preferred_element_type=jnp.float32) + b2_ref[...]


def _fc(hf, fc1_w, fc1_b, fc2_w, fc2_b, nblk=20):
    f1, kdim = fc1_w.shape
    ncls = fc2_w.shape[0]
    bk = kdim // nblk
    return pl.pallas_call(
        functools.partial(_fc_body, nblk),
        grid=(nblk,),
        in_specs=[
            pl.BlockSpec((bk, 1), lambda i: (i, 0)),
            pl.BlockSpec((f1, bk), lambda i: (0, i)),
            pl.BlockSpec((f1, 1), lambda i: (0, 0)),
            pl.BlockSpec((ncls, f1), lambda i: (0, 0)),
            pl.BlockSpec((1, ncls), lambda i: (0, 0)),
        ],
        out_specs=pl.BlockSpec((1, ncls), lambda i: (0, 0)),
        out_shape=jax.ShapeDtypeStruct((1, ncls), jnp.float32),
        scratch_shapes=[pltpu.VMEM((f1, 1), jnp.float32)],
    )(hf, fc1_w, fc1_b, fc2_w, fc2_b)


def kernel(x, edge_index, edge_weight, bn_gamma, bn_beta, bn_mean, bn_var,
           W0, W1, fc1_w, fc1_b, fc2_w, fc2_b):
    n, _ = x.shape
    hid = W0.shape[1]
    half = hid // 2
    scale = bn_gamma / jnp.sqrt(bn_var + 1e-5)
    shift = bn_beta - bn_mean * scale
    wlo = jnp.concatenate([W0[:, :half], W1[:, :half]], axis=1)
    whi = jnp.concatenate([W0[:, half:], W1[:, half:]], axis=1)
    t = _prep(x, scale.reshape(1, -1), shift.reshape(1, -1), wlo, whi)

    src = edge_index[0]
    dst = edge_index[1]
    e = src.shape[0]
    e_pad = e + (-e) % (_NS * _CHUNK)
    pad = e_pad - e
    src_p = jnp.concatenate([src, jnp.zeros((pad,), jnp.int32)])
    dst_p = jnp.concatenate([dst, jnp.ones((pad,), jnp.int32)])
    ew_p = jnp.concatenate([edge_weight, jnp.zeros((pad,), jnp.float32)])
    wm = (src_p == dst_p).astype(jnp.float32)
    wr = jnp.broadcast_to(
        jnp.stack([wm, ew_p], axis=1)[:, :, None],
        (e_pad, 2, 16)).reshape(e_pad * 32)
    wr = jnp.concatenate([wr, jnp.zeros((8 * 32 * _CHUNK,), jnp.float32)])

    acc = _sc_agg_call(n, hid, e_pad)(src_p, dst_p, wr, t)
    h = _relu(acc[:, :n, :])
    hf = jnp.concatenate([h[0], h[1]], axis=1).reshape(n * hid, 1)
    out = _fc(hf, fc1_w, fc1_b.reshape(-1, 1), fc2_w, fc2_b.reshape(1, -1))
    return out
